# baseline (speedup 1.0000x reference)
import functools

import jax
import jax.numpy as jnp
from jax import lax
from jax.experimental import pallas as pl
from jax.experimental.pallas import tpu as pltpu
from jax.experimental.pallas import tpu_sc as plsc

_M = 1024
_D = 256
_BLK = 2304


def _vq_kernel(nblocks, n_rows, x_ref, embt_ref, embt_bf_ref,
               inds_ref, cp_ref, pp_ref,
               e2_ref, psum_ref, hist_ref, riota_ref):
    i = pl.program_id(0)

    @pl.when(i == 0)
    def _init():
        embt = embt_ref[...]
        e2_ref[...] = jnp.sum(embt * embt, axis=0, keepdims=True)
        psum_ref[...] = jnp.zeros_like(psum_ref)
        hist_ref[...] = jnp.zeros_like(hist_ref)
        iota_i = jax.lax.broadcasted_iota(jnp.int32, riota_ref.shape, 1)
        riota_ref[...] = (_M - iota_i).astype(jnp.float32)

    x = x_ref[...]
    x2 = jnp.sum(x * x, axis=1, keepdims=True)
    s = jnp.dot(x.astype(jnp.bfloat16), embt_bf_ref[...],
                preferred_element_type=jnp.float32)
    dmap = 2.0 * s - (e2_ref[...] + x2)

    m = jnp.max(dmap, axis=1, keepdims=True)
    masked = jnp.where(dmap == m, riota_ref[...], 0.0)
    r = jnp.max(masked, axis=1, keepdims=True)
    k = (float(_M) - r).astype(jnp.int32)
    inds_ref[...] = k

    p = jnp.exp(dmap - m)
    probs = p / jnp.sum(p, axis=1, keepdims=True)
    psum_ref[...] += jnp.sum(probs, axis=0, keepdims=True)

    ohf = jnp.where(masked == r, 1.0, 0.0)
    hist_ref[...] += jnp.sum(ohf, axis=0, keepdims=True)

    @pl.when(i == nblocks - 1)
    def _finish():
        inv_n = 1.0 / n_rows
        hp = hist_ref[...] * inv_n
        cp_ref[...] = -jnp.sum(hp * (jnp.log2(hp + 1e-10)), axis=1, keepdims=True)
        ap = psum_ref[...] * inv_n
        pp_ref[...] = -jnp.sum(ap * (jnp.log2(ap + 1e-10)), axis=1, keepdims=True)


def _sc_gather(n, table_hbm, idx_hbm, out_hbm, idx_v, rows_v, sem):
    info = plsc.get_sparse_core_info()
    nw = info.num_cores * info.num_subcores
    b_per_w = n // nw
    chunk = 96
    wid = lax.axis_index("s") * info.num_cores + lax.axis_index("c")
    base = wid * b_per_w
    for c in range(b_per_w // chunk):
        pltpu.sync_copy(idx_hbm.at[pl.ds(base + c * chunk, chunk)], idx_v)
        pltpu.async_copy(table_hbm.at[idx_v], rows_v, sem).wait()
        pltpu.sync_copy(rows_v, out_hbm.at[pl.ds(base + c * chunk, chunk)])


def kernel(x, embedding):
    bsz, tsz, csz = x.shape
    n = bsz * tsz
    x_flat = x.reshape(n, csz)
    emb = embedding[0]
    embt = emb.T
    nblocks = n // _BLK

    inds, cp, pp = pl.pallas_call(
        functools.partial(_vq_kernel, nblocks, float(n)),
        grid=(nblocks,),
        in_specs=[
            pl.BlockSpec((_BLK, _D), lambda i: (i, 0)),
            pl.BlockSpec((_D, _M), lambda i: (0, 0)),
            pl.BlockSpec((_D, _M), lambda i: (0, 0)),
        ],
        out_specs=[
            pl.BlockSpec((_BLK, 1), lambda i: (i, 0)),
            pl.BlockSpec((1, 1), lambda i: (0, 0)),
            pl.BlockSpec((1, 1), lambda i: (0, 0)),
        ],
        out_shape=[
            jax.ShapeDtypeStruct((n, 1), jnp.int32),
            jax.ShapeDtypeStruct((1, 1), jnp.float32),
            jax.ShapeDtypeStruct((1, 1), jnp.float32),
        ],
        scratch_shapes=[
            pltpu.VMEM((1, _M), jnp.float32),
            pltpu.VMEM((1, _M), jnp.float32),
            pltpu.VMEM((1, _M), jnp.float32),
            pltpu.VMEM((1, _M), jnp.float32),
        ],
    )(x_flat, embt, embt.astype(jnp.bfloat16))

    table = emb.astype(jnp.bfloat16).astype(jnp.float32)
    mesh = plsc.VectorSubcoreMesh(core_axis_name="c", subcore_axis_name="s")
    chunk = 96
    q = pl.kernel(
        functools.partial(_sc_gather, n),
        mesh=mesh,
        out_type=jax.ShapeDtypeStruct((n, _D), jnp.float32),
        scratch_types=[
            pltpu.VMEM((chunk,), jnp.int32),
            pltpu.VMEM((chunk, _D), jnp.float32),
            pltpu.SemaphoreType.DMA,
        ],
    )(table, inds.reshape(n))

    quantized = q.reshape(bsz, tsz, csz)
    quantization_inds = inds.reshape(bsz, tsz, 1)
    return (quantized, cp[0, 0], pp[0, 0], quantization_inds)
